# baseline (speedup 1.0000x reference)
import dataclasses
import functools

import jax
import jax.numpy as jnp
from jax import lax
from jax.experimental import pallas as pl
from jax.experimental.pallas import tpu as pltpu
from jax.experimental.pallas import tpu_sc as plsc

NC = 2
NS = 16
NW = NC * NS
K = 128
LANES = 16


def _spmm_kernel(vals_hbm, src_hbm, dst_hbm, w_hbm, rank_hbm, out_hbm,
                 src_v, dst_v, w_v, rank_v, dstp_v, rows_v, zero_v, acc_sh, sem,
                 *, n_pad, n_chunks, d):
    cid = lax.axis_index("c")
    sid = lax.axis_index("s")
    wid = sid * NC + cid

    n_sub = n_pad // NS
    zr = zero_v.shape[0]

    @pl.loop(0, zr)
    def _(i):
        for j in range(d // LANES):
            zero_v[i, pl.ds(j * LANES, LANES)] = jnp.zeros((LANES,), jnp.float32)

    @pl.loop(0, n_sub // zr)
    def _(t):
        pltpu.sync_copy(zero_v, acc_sh.at[pl.ds(sid * n_sub + t * zr, zr)])

    plsc.subcore_barrier()

    dummy_base = n_pad - K

    @pl.loop(0, n_chunks)
    def _(c):
        pltpu.sync_copy(src_hbm.at[wid, c], src_v)
        pltpu.sync_copy(dst_hbm.at[wid, c], dst_v)
        pltpu.sync_copy(w_hbm.at[wid, c], w_v)
        pltpu.sync_copy(rank_hbm.at[wid, c], rank_v)
        pltpu.async_copy(vals_hbm.at[src_v], rows_v, sem).wait()

        @pl.loop(0, K)
        def _(i):
            wb = plsc.load_gather(w_v, [jnp.full((LANES,), 0, jnp.int32) + i])
            for j in range(d // LANES):
                sl = pl.ds(j * LANES, LANES)
                rows_v[i, sl] = rows_v[i, sl] * wb

        mr = rank_v[pl.ds(0, LANES)]
        for j in range(1, K // LANES):
            mr = jnp.maximum(mr, rank_v[pl.ds(j * LANES, LANES)])
        max_rank = jax.lax.reduce_max(mr, axes=(0,))

        @pl.loop(0, max_rank + 1)
        def _(r):
            for j in range(K // LANES):
                sl = pl.ds(j * LANES, LANES)
                dummies = lax.iota(jnp.int32, LANES) + (dummy_base + j * LANES)
                dstp_v[sl] = jnp.where(rank_v[sl] == r, dst_v[sl], dummies)
            pltpu.sync_copy(rows_v, acc_sh.at[dstp_v], add=True)

    plsc.subcore_barrier()

    pltpu.sync_copy(acc_sh.at[pl.ds(sid * n_sub, n_sub)],
                    out_hbm.at[cid, pl.ds(sid * n_sub, n_sub)])


def _spmm(vals, srcg, dstg, wg, rankg):
    n_nodes, d = vals.shape
    n_pad = -(-(n_nodes + K) // (8 * NS)) * (8 * NS)
    n_chunks = srcg.shape[1]
    mesh = plsc.VectorSubcoreMesh(core_axis_name="c", subcore_axis_name="s",
                                  num_cores=NC, num_subcores=NS)
    zr = 128
    body = functools.partial(_spmm_kernel, n_pad=n_pad, n_chunks=n_chunks, d=d)
    cp = pltpu.CompilerParams()
    if "needs_layout_passes" in pltpu.CompilerParams.__dataclass_fields__:
        cp = dataclasses.replace(cp, needs_layout_passes=False)
    return pl.kernel(
        body,
        compiler_params=cp,
        out_type=jax.ShapeDtypeStruct((NC, n_pad, d), jnp.float32),
        mesh=mesh,
        scratch_types=[
            pltpu.VMEM((K,), jnp.int32),
            pltpu.VMEM((K,), jnp.int32),
            pltpu.VMEM((K,), jnp.float32),
            pltpu.VMEM((K,), jnp.int32),
            pltpu.VMEM((K,), jnp.int32),
            pltpu.VMEM((K, d), jnp.float32),
            pltpu.VMEM((zr, d), jnp.float32),
            pltpu.VMEM_SHARED((n_pad, d), jnp.float32),
            pltpu.SemaphoreType.DMA,
        ],
    )(vals, srcg, dstg, wg, rankg)


def _mid_body(p_ref, w1_ref, b1_ref, w2_ref, t_ref):
    agg = p_ref[0] + p_ref[1]
    h = jnp.dot(agg, w1_ref[...], preferred_element_type=jnp.float32)
    h = jnp.maximum(h + b1_ref[...], 0.0)
    t_ref[...] = jnp.dot(h, w2_ref[...], preferred_element_type=jnp.float32)


def _fin_body(p_ref, b2_ref, o_ref):
    o_ref[...] = p_ref[0] + p_ref[1] + b2_ref[...]


def _dense_mid(p, w1, b1, w2, n):
    blk = 1000
    grid = (n // blk,)
    return pl.pallas_call(
        _mid_body,
        grid=grid,
        in_specs=[
            pl.BlockSpec((2, blk, p.shape[2]), lambda i: (0, i, 0)),
            pl.BlockSpec(w1.shape, lambda i: (0, 0)),
            pl.BlockSpec((1, b1.shape[1]), lambda i: (0, 0)),
            pl.BlockSpec(w2.shape, lambda i: (0, 0)),
        ],
        out_specs=pl.BlockSpec((blk, w2.shape[1]), lambda i: (i, 0)),
        out_shape=jax.ShapeDtypeStruct((n, w2.shape[1]), jnp.float32),
    )(p, w1, b1, w2)


def _dense_fin(p, b2, n):
    blk = 1000
    grid = (n // blk,)
    return pl.pallas_call(
        _fin_body,
        grid=grid,
        in_specs=[
            pl.BlockSpec((2, blk, p.shape[2]), lambda i: (0, i, 0)),
            pl.BlockSpec((1, b2.shape[1]), lambda i: (0, 0)),
        ],
        out_specs=pl.BlockSpec((blk, p.shape[2]), lambda i: (i, 0)),
        out_shape=jax.ShapeDtypeStruct((n, p.shape[2]), jnp.float32),
    )(p, b2)


@jax.jit
def _gcn(x, edge_index, edge_weight, W1, b1, W2, b2):
    e = edge_index.shape[1]
    n_chunks = -(-e // (NW * K))
    epad = NW * K * n_chunks
    pad = epad - e

    src = edge_index[0].astype(jnp.int32)
    dst = edge_index[1].astype(jnp.int32)
    w = edge_weight.astype(jnp.float32)
    if pad:
        src = jnp.concatenate([src, jnp.zeros((pad,), jnp.int32)])
        dst = jnp.concatenate([dst, jnp.zeros((pad,), jnp.int32)])
        w = jnp.concatenate([w, jnp.zeros((pad,), jnp.float32)])
    srcg = src.reshape(NW, n_chunks, K)
    dstg = dst.reshape(NW, n_chunks, K)
    wg = w.reshape(NW, n_chunks, K)

    dstc = dst.reshape(-1, K)
    eq = dstc[:, :, None] == dstc[:, None, :]
    tril = jnp.tril(jnp.ones((K, K), bool), -1)
    rankg = jnp.sum(eq & tril[None], axis=2, dtype=jnp.int32).reshape(NW, n_chunks, K)

    n = x.shape[0]
    p1 = _spmm(x, srcg, dstg, wg, rankg)
    t = _dense_mid(p1, W1, b1.reshape(1, -1), W2, n)
    p2 = _spmm(t, srcg, dstg, wg, rankg)
    return _dense_fin(p2, b2.reshape(1, -1), n)


def kernel(x, edge_index, edge_weight, W1, b1, W2, b2):
    return _gcn(x, edge_index, edge_weight, W1, b1, W2, b2)


# speedup vs baseline: 1.1687x; 1.1687x over previous
import dataclasses
import functools

import jax
import jax.numpy as jnp
from jax import lax
from jax.experimental import pallas as pl
from jax.experimental.pallas import tpu as pltpu
from jax.experimental.pallas import tpu_sc as plsc

NC = 2
NS = 16
NW = NC * NS
K = 128
LANES = 16


def _spmm_kernel(vals_hbm, idx_hbm, out_hbm,
                 idx0_v, idx1_v, dst_v, rank_v, dstp_v, rows0_v, rows1_v,
                 acc_sh, gsem0, gsem1, isem0, isem1,
                 *, n_pad, n_chunks, d):
    cid = lax.axis_index("c")
    sid = lax.axis_index("s")
    wid = sid * NC + cid

    n_sub = n_pad // NS
    zr = rows0_v.shape[0]

    @pl.loop(0, zr)
    def _(i):
        for j in range(d // LANES):
            rows0_v[i, pl.ds(j * LANES, LANES)] = jnp.zeros((LANES,), jnp.float32)

    @pl.loop(0, n_sub // zr)
    def _(t):
        pltpu.sync_copy(rows0_v, acc_sh.at[pl.ds(sid * n_sub + t * zr, zr)])

    plsc.subcore_barrier()

    dummy_base = n_pad - K
    last_c = n_chunks - 1

    def idx_issue(c, idx_v, sem):
        pltpu.async_copy(idx_hbm.at[wid, c], idx_v, sem)

    def idx_wait(idx_v, sem):
        pltpu.make_async_copy(idx_hbm.at[0, 0], idx_v, sem).wait()

    def gather_issue(idx_v, rows_v, sem):
        pltpu.async_copy(vals_hbm.at[idx_v.at[0]], rows_v, sem)

    def gather_wait(rows_v, sem):
        pltpu.make_async_copy(vals_hbm.at[idx0_v.at[0]], rows_v, sem).wait()

    def unpack(idx_v):
        for j in range(K // LANES):
            sl = pl.ds(j * LANES, LANES)
            dr = idx_v[1, sl]
            dst_v[sl] = dr & 0xFFFF
            rank_v[sl] = lax.shift_right_logical(dr, 16)

    def scale(idx_v, rows_v):
        @pl.loop(0, K)
        def _(i):
            wb = plsc.load_gather(
                idx_v, [jnp.full((LANES,), 2, jnp.int32),
                        jnp.full((LANES,), 0, jnp.int32) + i])
            wb = plsc.bitcast(wb, jnp.float32)
            for j in range(d // LANES):
                sl = pl.ds(j * LANES, LANES)
                rows_v[i, sl] = rows_v[i, sl] * wb

    def scatter(rows_v):
        mr = rank_v[pl.ds(0, LANES)]
        for j in range(1, K // LANES):
            mr = jnp.maximum(mr, rank_v[pl.ds(j * LANES, LANES)])
        max_rank = jax.lax.reduce_max(mr, axes=(0,))

        @pl.loop(0, max_rank + 1)
        def _(r):
            for j in range(K // LANES):
                sl = pl.ds(j * LANES, LANES)
                dummies = lax.iota(jnp.int32, LANES) + (dummy_base + j * LANES)
                dstp_v[sl] = jnp.where(rank_v[sl] == r, dst_v[sl], dummies)
            pltpu.sync_copy(rows_v, acc_sh.at[dstp_v], add=True)

    def consume(idx_v, rows_v, sem):
        gather_wait(rows_v, sem)
        unpack(idx_v)
        scale(idx_v, rows_v)
        scatter(rows_v)

    pltpu.sync_copy(idx_hbm.at[wid, 0], idx0_v)
    gather_issue(idx0_v, rows0_v, gsem0)

    @pl.loop(0, n_chunks // 2 - 1)
    def _(p):
        c0 = 2 * p
        pltpu.sync_copy(idx_hbm.at[wid, c0 + 1], idx1_v)
        gather_issue(idx1_v, rows1_v, gsem1)
        consume(idx0_v, rows0_v, gsem0)
        pltpu.sync_copy(idx_hbm.at[wid, c0 + 2], idx0_v)
        gather_issue(idx0_v, rows0_v, gsem0)
        consume(idx1_v, rows1_v, gsem1)

    pltpu.sync_copy(idx_hbm.at[wid, n_chunks - 1], idx1_v)
    gather_issue(idx1_v, rows1_v, gsem1)
    consume(idx0_v, rows0_v, gsem0)
    consume(idx1_v, rows1_v, gsem1)

    plsc.subcore_barrier()

    pltpu.sync_copy(acc_sh.at[pl.ds(sid * n_sub, n_sub)],
                    out_hbm.at[cid, pl.ds(sid * n_sub, n_sub)])


def _spmm(vals, idxg):
    n_nodes, d = vals.shape
    n_pad = -(-(n_nodes + K) // (8 * NS)) * (8 * NS)
    n_chunks = idxg.shape[1]
    mesh = plsc.VectorSubcoreMesh(core_axis_name="c", subcore_axis_name="s",
                                  num_cores=NC, num_subcores=NS)
    body = functools.partial(_spmm_kernel, n_pad=n_pad, n_chunks=n_chunks, d=d)
    cp = pltpu.CompilerParams()
    if "needs_layout_passes" in pltpu.CompilerParams.__dataclass_fields__:
        cp = dataclasses.replace(cp, needs_layout_passes=False)
    return pl.kernel(
        body,
        compiler_params=cp,
        out_type=jax.ShapeDtypeStruct((NC, n_pad, d), jnp.float32),
        mesh=mesh,
        scratch_types=[
            pltpu.VMEM((3, K), jnp.int32),
            pltpu.VMEM((3, K), jnp.int32),
            pltpu.VMEM((K,), jnp.int32),
            pltpu.VMEM((K,), jnp.int32),
            pltpu.VMEM((K,), jnp.int32),
            pltpu.VMEM((K, d), jnp.float32),
            pltpu.VMEM((K, d), jnp.float32),
            pltpu.VMEM_SHARED((n_pad, d), jnp.float32),
            pltpu.SemaphoreType.DMA,
            pltpu.SemaphoreType.DMA,
            pltpu.SemaphoreType.DMA,
            pltpu.SemaphoreType.DMA,
        ],
    )(vals, idxg)


def _mid_body(p_ref, w1_ref, b1_ref, w2_ref, t_ref):
    agg = p_ref[0] + p_ref[1]
    h = jnp.dot(agg, w1_ref[...], preferred_element_type=jnp.float32)
    h = jnp.maximum(h + b1_ref[...], 0.0)
    t_ref[...] = jnp.dot(h, w2_ref[...], preferred_element_type=jnp.float32)


def _fin_body(p_ref, b2_ref, o_ref):
    o_ref[...] = p_ref[0] + p_ref[1] + b2_ref[...]


def _dense_mid(p, w1, b1, w2, n):
    blk = 1000
    grid = (n // blk,)
    return pl.pallas_call(
        _mid_body,
        grid=grid,
        in_specs=[
            pl.BlockSpec((2, blk, p.shape[2]), lambda i: (0, i, 0)),
            pl.BlockSpec(w1.shape, lambda i: (0, 0)),
            pl.BlockSpec((1, b1.shape[1]), lambda i: (0, 0)),
            pl.BlockSpec(w2.shape, lambda i: (0, 0)),
        ],
        out_specs=pl.BlockSpec((blk, w2.shape[1]), lambda i: (i, 0)),
        out_shape=jax.ShapeDtypeStruct((n, w2.shape[1]), jnp.float32),
    )(p, w1, b1, w2)


def _dense_fin(p, b2, n):
    blk = 1000
    grid = (n // blk,)
    return pl.pallas_call(
        _fin_body,
        grid=grid,
        in_specs=[
            pl.BlockSpec((2, blk, p.shape[2]), lambda i: (0, i, 0)),
            pl.BlockSpec((1, b2.shape[1]), lambda i: (0, 0)),
        ],
        out_specs=pl.BlockSpec((blk, p.shape[2]), lambda i: (i, 0)),
        out_shape=jax.ShapeDtypeStruct((n, p.shape[2]), jnp.float32),
    )(p, b2)


@jax.jit
def _gcn(x, edge_index, edge_weight, W1, b1, W2, b2):
    e = edge_index.shape[1]
    n_chunks = -(-e // (NW * K))
    epad = NW * K * n_chunks
    pad = epad - e

    src = edge_index[0].astype(jnp.int32)
    dst = edge_index[1].astype(jnp.int32)
    w = edge_weight.astype(jnp.float32)
    if pad:
        src = jnp.concatenate([src, jnp.zeros((pad,), jnp.int32)])
        dst = jnp.concatenate([dst, jnp.zeros((pad,), jnp.int32)])
        w = jnp.concatenate([w, jnp.zeros((pad,), jnp.float32)])
    dstc = dst.reshape(-1, K)
    eq = dstc[:, :, None] == dstc[:, None, :]
    tril = jnp.tril(jnp.ones((K, K), bool), -1)
    rank = jnp.sum(eq & tril[None], axis=2, dtype=jnp.int32).reshape(-1)

    wbits = lax.bitcast_convert_type(w, jnp.int32)
    dstrank = dst | (rank << 16)
    idxg = jnp.stack([src.reshape(NW, n_chunks, K),
                      dstrank.reshape(NW, n_chunks, K),
                      wbits.reshape(NW, n_chunks, K)], axis=2)

    n = x.shape[0]
    p1 = _spmm(x, idxg)
    t = _dense_mid(p1, W1, b1.reshape(1, -1), W2, n)
    p2 = _spmm(t, idxg)
    return _dense_fin(p2, b2.reshape(1, -1), n)


def kernel(x, edge_index, edge_weight, W1, b1, W2, b2):
    return _gcn(x, edge_index, edge_weight, W1, b1, W2, b2)
